# baseline (speedup 1.0000x reference)
import functools

import jax
import jax.numpy as jnp
from jax import lax
from jax.experimental import pallas as pl
from jax.experimental.pallas import tpu as pltpu
from jax.experimental.pallas import tpu_sc as plsc

_NUM_USERS = 4000
_NUM_ITEMS = 6000
_F = 128
_N = _NUM_USERS + _NUM_ITEMS
_E = 320000
_LAYERS = 3

_NC = 2
_NS = 16
_W = _NC * _NS
_EPW = _E // _W
_K = 80
_NCHUNK = _EPW // _K
_NP = 10240
_RPT = _NP // _NS
_NH = 10240
_HPT = _NH // _NS

@functools.cache
def _get_mesh():
    return plsc.VectorSubcoreMesh(core_axis_name="c", subcore_axis_name="s",
                                  num_cores=_NC, num_subcores=_NS)


def _hist_body(src_hbm, dst_hbm, zeros_hbm, out_hbm,
               hist_r, hist_c, sidx2, didx2, ones_v, sem_r, sem_c):
    ci = lax.axis_index("c")
    si = lax.axis_index("s")
    wid = ci * _NS + si

    ones16 = jnp.full((16,), 1.0, dtype=jnp.float32)
    for j in range(_K // 16):
        ones_v[pl.ds(j * 16, 16)] = ones16

    pltpu.sync_copy(zeros_hbm.at[pl.ds(si * _HPT, _HPT)],
                    hist_r.at[pl.ds(si * _HPT, _HPT)])
    pltpu.sync_copy(zeros_hbm.at[pl.ds(si * _HPT, _HPT)],
                    hist_c.at[pl.ds(si * _HPT, _HPT)])
    pltpu.sync_copy(src_hbm.at[wid], sidx2)
    pltpu.sync_copy(dst_hbm.at[wid], didx2)
    plsc.subcore_barrier()

    def fire(t):
        pltpu.async_copy(ones_v, hist_r.at[didx2.at[t]], sem_r, add=True)
        pltpu.async_copy(ones_v, hist_c.at[sidx2.at[t]], sem_c, add=True)

    def drain(t):
        pltpu.make_async_copy(ones_v, hist_r.at[didx2.at[t]], sem_r).wait()
        pltpu.make_async_copy(ones_v, hist_c.at[sidx2.at[t]], sem_c).wait()

    fire(0)

    def chunk(t, carry):
        fire(t)
        drain(t - 1)
        return carry

    lax.fori_loop(1, _NCHUNK, chunk, 0)
    drain(_NCHUNK - 1)
    plsc.subcore_barrier()

    pltpu.sync_copy(hist_r.at[pl.ds(si * _HPT, _HPT)],
                    out_hbm.at[pl.ds(2 * ci * _NH + si * _HPT, _HPT)])
    pltpu.sync_copy(hist_c.at[pl.ds(si * _HPT, _HPT)],
                    out_hbm.at[pl.ds((2 * ci + 1) * _NH + si * _HPT, _HPT)])


@functools.cache
def _get_hist():
    return pl.kernel(
        _hist_body,
        out_type=jax.ShapeDtypeStruct((4 * _NH,), jnp.float32),
        mesh=_get_mesh(),
        scratch_types=[
            pltpu.VMEM_SHARED((_NH,), jnp.float32),
            pltpu.VMEM_SHARED((_NH,), jnp.float32),
            pltpu.VMEM((_NCHUNK, _K), jnp.int32),
            pltpu.VMEM((_NCHUNK, _K), jnp.int32),
            pltpu.VMEM((_K,), jnp.float32),
            pltpu.SemaphoreType.DMA,
            pltpu.SemaphoreType.DMA,
        ],
    )


_NBUF = 2


def _spmm_body(gath_hbm, src_hbm, dst_hbm, zeros_hbm, out_hbm,
               acc_sh, sidx_v, didx_v, *bufs_and_sems):
    rows = bufs_and_sems[:_NBUF]
    gsem = bufs_and_sems[_NBUF:2 * _NBUF]
    ssem = bufs_and_sems[2 * _NBUF:]
    ci = lax.axis_index("c")
    si = lax.axis_index("s")
    wid = ci * _NS + si

    pltpu.sync_copy(zeros_hbm.at[pl.ds(si * _RPT, _RPT)],
                    acc_sh.at[pl.ds(si * _RPT, _RPT)])
    pltpu.sync_copy(src_hbm.at[pl.ds(wid * _EPW, _EPW)], sidx_v)
    pltpu.sync_copy(dst_hbm.at[wid], didx_v)
    plsc.subcore_barrier()

    def fire_gather(t, b):
        tc = jnp.minimum(t, _NCHUNK - 1)
        pltpu.async_copy(gath_hbm.at[sidx_v.at[pl.ds(tc * _K, _K)]],
                         rows[b], gsem[b])

    def wait_gather(t, b):
        tc = jnp.minimum(t, _NCHUNK - 1)
        pltpu.make_async_copy(gath_hbm.at[sidx_v.at[pl.ds(tc * _K, _K)]],
                              rows[b], gsem[b]).wait()

    def fire_scatter(t, b):
        pltpu.async_copy(rows[b], acc_sh.at[didx_v.at[t]], ssem[b], add=True)

    def wait_scatter(t, b):
        pltpu.make_async_copy(rows[b], acc_sh.at[didx_v.at[t]],
                              ssem[b]).wait()

    for b in range(_NBUF):
        fire_gather(b, b)

    def quad(i, carry):
        t0 = _NBUF * i
        for b in range(_NBUF):
            wait_gather(t0 + b, b)
            fire_scatter(t0 + b, b)
        for b in range(_NBUF):
            wait_scatter(t0 + b, b)
            fire_gather(t0 + _NBUF + b, b)
        return carry

    lax.fori_loop(0, _NCHUNK // _NBUF, quad, 0)
    t_last = _NCHUNK - 1
    wait_gather(t_last, 0)
    pltpu.sync_copy(rows[0], acc_sh.at[didx_v.at[t_last]], add=True)
    for b in range(1, _NBUF):
        wait_gather(t_last + b, b)
    plsc.subcore_barrier()

    pltpu.sync_copy(acc_sh.at[pl.ds(si * _RPT, _RPT)],
                    out_hbm.at[pl.ds(ci * _NP + si * _RPT, _RPT)])


@functools.cache
def _get_spmm():
    return pl.kernel(
        _spmm_body,
        out_type=jax.ShapeDtypeStruct((2 * _NP, _F), jnp.float32),
        mesh=_get_mesh(),
        scratch_types=(
            [pltpu.VMEM_SHARED((_NP, _F), jnp.float32),
             pltpu.VMEM((_EPW,), jnp.int32),
             pltpu.VMEM((_NCHUNK, _K), jnp.int32)]
            + [pltpu.VMEM((_K, _F), jnp.float32)] * _NBUF
            + [pltpu.SemaphoreType.DMA] * (2 * _NBUF)
        ),
    )


def _scale_body(hp_ref, out_ref):
    hp = hp_ref[...]
    r = lax.rsqrt(jnp.maximum(hp[0] + hp[2], 1.0))
    c = lax.rsqrt(jnp.maximum(hp[1] + hp[3], 1.0))
    out_ref[...] = jnp.stack([r, c, r * c])


_scale = pl.pallas_call(
    _scale_body,
    out_shape=jax.ShapeDtypeStruct((3, _NH), jnp.float32),
)


_EWB = 1000


def _ew_body(p0_ref, p1_ref, acc_ref, rv_ref, rcv_ref, accn_ref, g_ref):
    comb = p0_ref[...] + p1_ref[...]
    accn_ref[...] = acc_ref[...] + rv_ref[...] * comb
    g_ref[...] = rcv_ref[...] * comb


def _ew_last_body(p0_ref, p1_ref, acc_ref, rv_ref, accn_ref):
    accn_ref[...] = acc_ref[...] + rv_ref[...] * (p0_ref[...] + p1_ref[...])


_row_spec = pl.BlockSpec((_EWB, _F), lambda i: (i, 0))
_col_spec = pl.BlockSpec((_EWB, 1), lambda i: (i, 0))

_ew = pl.pallas_call(
    _ew_body,
    grid=(_N // _EWB,),
    in_specs=[_row_spec, _row_spec, _row_spec, _col_spec, _col_spec],
    out_specs=[_row_spec, _row_spec],
    out_shape=[jax.ShapeDtypeStruct((_N, _F), jnp.float32),
               jax.ShapeDtypeStruct((_N, _F), jnp.float32)],
)

_ew_last = pl.pallas_call(
    _ew_last_body,
    grid=(_N // _EWB,),
    in_specs=[_row_spec, _row_spec, _row_spec, _col_spec],
    out_specs=_row_spec,
    out_shape=jax.ShapeDtypeStruct((_N, _F), jnp.float32),
)


def kernel(edge_index, user_emb, item_emb):
    src = edge_index[0]
    dst = edge_index[1]
    src_r = src.reshape(_W, _NCHUNK, _K)
    dst_r = dst.reshape(_W, _NCHUNK, _K)
    feat = jnp.concatenate([user_emb, item_emb], axis=0)
    zeros_nh = jnp.zeros((_NH,), jnp.float32)
    zeros_nf = jnp.zeros((_N, _F), jnp.float32)
    zeros_npf = jnp.zeros((_NP, _F), jnp.float32)
    ones_col = jnp.ones((_N, 1), jnp.float32)

    hp = _get_hist()(src_r, dst_r, zeros_nh).reshape(4, _NH)
    scl = _scale(hp)
    r_col = scl[0, :_N, None]
    c_col = scl[1, :_N, None]
    rc_col = scl[2, :_N, None]

    acc, gath = _ew(feat, zeros_nf, zeros_nf, ones_col, c_col)
    for layer in range(_LAYERS):
        p = _get_spmm()(gath, src, dst_r, zeros_npf)
        p0 = p[:_N]
        p1 = p[_NP:_NP + _N]
        if layer < _LAYERS - 1:
            acc, gath = _ew(p0, p1, acc, r_col, rc_col)
        else:
            acc = _ew_last(p0, p1, acc, r_col)

    return acc[:_NUM_USERS], acc[_NUM_USERS:]


# speedup vs baseline: 1.2110x; 1.2110x over previous
import functools

import jax
import jax.numpy as jnp
from jax import lax
from jax.experimental import pallas as pl
from jax.experimental.pallas import tpu as pltpu
from jax.experimental.pallas import tpu_sc as plsc

_NUM_USERS = 4000
_NUM_ITEMS = 6000
_F = 128
_N = _NUM_USERS + _NUM_ITEMS
_E = 320000
_LAYERS = 3

_NC = 2
_NS = 16
_W = _NC * _NS
_EPW = _E // _W
_K = 80
_NCHUNK = _EPW // _K
_NP = 10240
_RPT = _NP // _NS
_NH = 10240
_HPT = _NH // _NS

@functools.cache
def _get_mesh():
    return plsc.VectorSubcoreMesh(core_axis_name="c", subcore_axis_name="s",
                                  num_cores=_NC, num_subcores=_NS)


def _hist_body(src_hbm, dst_hbm, zeros_hbm, out_hbm,
               hist_r, hist_c, sidx2, didx2, ones_v, sem_r, sem_c):
    ci = lax.axis_index("c")
    si = lax.axis_index("s")
    wid = ci * _NS + si

    ones16 = jnp.full((16,), 1.0, dtype=jnp.float32)
    for j in range(_K // 16):
        ones_v[pl.ds(j * 16, 16)] = ones16

    pltpu.sync_copy(zeros_hbm.at[pl.ds(si * _HPT, _HPT)],
                    hist_r.at[pl.ds(si * _HPT, _HPT)])
    pltpu.sync_copy(zeros_hbm.at[pl.ds(si * _HPT, _HPT)],
                    hist_c.at[pl.ds(si * _HPT, _HPT)])
    pltpu.sync_copy(src_hbm.at[wid], sidx2)
    pltpu.sync_copy(dst_hbm.at[wid], didx2)
    plsc.subcore_barrier()

    def fire(t):
        pltpu.async_copy(ones_v, hist_r.at[didx2.at[t]], sem_r, add=True)
        pltpu.async_copy(ones_v, hist_c.at[sidx2.at[t]], sem_c, add=True)

    def drain(t):
        pltpu.make_async_copy(ones_v, hist_r.at[didx2.at[t]], sem_r).wait()
        pltpu.make_async_copy(ones_v, hist_c.at[sidx2.at[t]], sem_c).wait()

    fire(0)

    def chunk(t, carry):
        fire(t)
        drain(t - 1)
        return carry

    lax.fori_loop(1, _NCHUNK, chunk, 0)
    drain(_NCHUNK - 1)
    plsc.subcore_barrier()

    pltpu.sync_copy(hist_r.at[pl.ds(si * _HPT, _HPT)],
                    out_hbm.at[pl.ds(2 * ci * _NH + si * _HPT, _HPT)])
    pltpu.sync_copy(hist_c.at[pl.ds(si * _HPT, _HPT)],
                    out_hbm.at[pl.ds((2 * ci + 1) * _NH + si * _HPT, _HPT)])


@functools.cache
def _get_hist():
    return pl.kernel(
        _hist_body,
        out_type=jax.ShapeDtypeStruct((4 * _NH,), jnp.float32),
        mesh=_get_mesh(),
        scratch_types=[
            pltpu.VMEM_SHARED((_NH,), jnp.float32),
            pltpu.VMEM_SHARED((_NH,), jnp.float32),
            pltpu.VMEM((_NCHUNK, _K), jnp.int32),
            pltpu.VMEM((_NCHUNK, _K), jnp.int32),
            pltpu.VMEM((_K,), jnp.float32),
            pltpu.SemaphoreType.DMA,
            pltpu.SemaphoreType.DMA,
        ],
    )


def _spmm_body(gath_hbm, src_hbm, dst_hbm, zeros_hbm, out_hbm,
               acc_sh, sidx_v, didx_v, rows0, rows1, gsem0, gsem1):
    ci = lax.axis_index("c")
    si = lax.axis_index("s")
    wid = ci * _NS + si

    pltpu.sync_copy(zeros_hbm.at[pl.ds(si * _RPT, _RPT)],
                    acc_sh.at[pl.ds(si * _RPT, _RPT)])
    pltpu.sync_copy(src_hbm.at[pl.ds(wid * _EPW, _EPW)], sidx_v)
    pltpu.sync_copy(dst_hbm.at[wid], didx_v)
    plsc.subcore_barrier()

    def gather(t, buf, sem):
        return pltpu.async_copy(gath_hbm.at[sidx_v.at[pl.ds(t * _K, _K)]],
                                buf, sem)

    def scatter(t, buf):
        pltpu.sync_copy(buf, acc_sh.at[didx_v.at[t]], add=True)

    gather(0, rows0, gsem0)

    def pair(i, carry):
        t0 = 2 * i
        gather(t0 + 1, rows1, gsem1)
        pltpu.make_async_copy(gath_hbm.at[sidx_v.at[pl.ds(t0 * _K, _K)]],
                              rows0, gsem0).wait()
        scatter(t0, rows0)
        gather(t0 + 2, rows0, gsem0)
        pltpu.make_async_copy(gath_hbm.at[sidx_v.at[pl.ds((t0 + 1) * _K, _K)]],
                              rows1, gsem1).wait()
        scatter(t0 + 1, rows1)
        return carry

    lax.fori_loop(0, (_NCHUNK - 1) // 2, pair, 0)
    t_last = _NCHUNK - 1
    pltpu.make_async_copy(gath_hbm.at[sidx_v.at[pl.ds(t_last * _K, _K)]],
                          rows0, gsem0).wait()
    scatter(t_last, rows0)
    plsc.subcore_barrier()

    pltpu.sync_copy(acc_sh.at[pl.ds(si * _RPT, _RPT)],
                    out_hbm.at[pl.ds(ci * _NP + si * _RPT, _RPT)])


@functools.cache
def _get_spmm():
    return pl.kernel(
        _spmm_body,
        out_type=jax.ShapeDtypeStruct((2 * _NP, _F), jnp.float32),
        mesh=_get_mesh(),
        scratch_types=[
            pltpu.VMEM_SHARED((_NP, _F), jnp.float32),
            pltpu.VMEM((_EPW,), jnp.int32),
            pltpu.VMEM((_NCHUNK, _K), jnp.int32),
            pltpu.VMEM((_K, _F), jnp.float32),
            pltpu.VMEM((_K, _F), jnp.float32),
            pltpu.SemaphoreType.DMA,
            pltpu.SemaphoreType.DMA,
        ],
    )


def _scale_body(hp_ref, out_ref):
    hp = hp_ref[...]
    r = lax.rsqrt(jnp.maximum(hp[0] + hp[2], 1.0))
    c = lax.rsqrt(jnp.maximum(hp[1] + hp[3], 1.0))
    out_ref[...] = jnp.stack([r, c, r * c])


_scale = pl.pallas_call(
    _scale_body,
    out_shape=jax.ShapeDtypeStruct((3, _NH), jnp.float32),
)


_EWB = 1000


def _ew_body(p0_ref, p1_ref, acc_ref, rv_ref, rcv_ref, accn_ref, g_ref):
    comb = p0_ref[...] + p1_ref[...]
    accn_ref[...] = acc_ref[...] + rv_ref[...] * comb
    g_ref[...] = rcv_ref[...] * comb


def _ew_last_body(p0_ref, p1_ref, acc_ref, rv_ref, accn_ref):
    accn_ref[...] = acc_ref[...] + rv_ref[...] * (p0_ref[...] + p1_ref[...])


_row_spec = pl.BlockSpec((_EWB, _F), lambda i: (i, 0))
_col_spec = pl.BlockSpec((_EWB, 1), lambda i: (i, 0))

_ew = pl.pallas_call(
    _ew_body,
    grid=(_N // _EWB,),
    in_specs=[_row_spec, _row_spec, _row_spec, _col_spec, _col_spec],
    out_specs=[_row_spec, _row_spec],
    out_shape=[jax.ShapeDtypeStruct((_N, _F), jnp.float32),
               jax.ShapeDtypeStruct((_N, _F), jnp.float32)],
)

_ew_last = pl.pallas_call(
    _ew_last_body,
    grid=(_N // _EWB,),
    in_specs=[_row_spec, _row_spec, _row_spec, _col_spec],
    out_specs=_row_spec,
    out_shape=jax.ShapeDtypeStruct((_N, _F), jnp.float32),
)


def kernel(edge_index, user_emb, item_emb):
    src = edge_index[0]
    dst = edge_index[1]
    src_r = src.reshape(_W, _NCHUNK, _K)
    dst_r = dst.reshape(_W, _NCHUNK, _K)
    feat = jnp.concatenate([user_emb, item_emb], axis=0)
    zeros_nh = jnp.zeros((_NH,), jnp.float32)
    zeros_nf = jnp.zeros((_N, _F), jnp.float32)
    zeros_npf = jnp.zeros((_NP, _F), jnp.float32)
    ones_col = jnp.ones((_N, 1), jnp.float32)

    hp = _get_hist()(src_r, dst_r, zeros_nh).reshape(4, _NH)
    scl = _scale(hp)
    r_col = scl[0, :_N, None]
    c_col = scl[1, :_N, None]
    rc_col = scl[2, :_N, None]

    acc, gath = _ew(feat, zeros_nf, zeros_nf, ones_col, c_col)
    for layer in range(_LAYERS):
        p = _get_spmm()(gath, src, dst_r, zeros_npf)
        p0 = p[:_N]
        p1 = p[_NP:_NP + _N]
        if layer < _LAYERS - 1:
            acc, gath = _ew(p0, p1, acc, r_col, rc_col)
        else:
            acc = _ew_last(p0, p1, acc, r_col)

    return acc[:_NUM_USERS], acc[_NUM_USERS:]
